# baseline (speedup 1.0000x reference)
import functools

import jax
import jax.numpy as jnp
from jax import lax
from jax.experimental import pallas as pl
from jax.experimental.pallas import tpu as pltpu
from jax.experimental.pallas import tpu_sc as plsc

NU = 6000
NI = 4000
NN = 10000
NE = 320000
D = 128
ALPHA = 0.8
BETA = 1.0

CH = 125
ROWS_E = NE // CH
NTILE = 16
NZ = NN // NTILE


def _sc_mesh():
    return plsc.VectorSubcoreMesh(core_axis_name="c", subcore_axis_name="s")


_SC_PARAMS = pltpu.CompilerParams(use_tc_tiling_on_sc=False)


@functools.cache
def _get_deg_kernel():
    @functools.partial(
        pl.kernel,
        out_type=jax.ShapeDtypeStruct((2 * NN, 16), jnp.float32),
        mesh=_sc_mesh(),
        scratch_types=[
            pltpu.VMEM((ROWS_E // 32, CH), jnp.int32),
            pltpu.VMEM((CH, 16), jnp.float32),
            pltpu.VMEM((NZ, 16), jnp.float32),
            pltpu.VMEM_SHARED((NN, 16), jnp.float32),
        ],
        compiler_params=_SC_PARAMS,
    )
    def _deg_kernel(dst_hbm, out_hbm, idx_v, ones_v, zb_v, acc_sh):
        c = lax.axis_index("c")
        s = lax.axis_index("s")
        nrow_t = ROWS_E // 32
        z16 = jnp.zeros((16,), jnp.float32)
        o16 = jnp.full((16,), 1.0, jnp.float32)

        def zb_body(i, _):
            zb_v[i, :] = z16
            return 0

        lax.fori_loop(0, NZ, zb_body, 0)

        def on_body(i, _):
            ones_v[i, :] = o16
            return 0

        lax.fori_loop(0, CH, on_body, 0)
        pltpu.sync_copy(zb_v, acc_sh.at[pl.ds(s * NZ, NZ)])
        plsc.subcore_barrier()

        base = (c * NTILE + s) * nrow_t
        pltpu.sync_copy(dst_hbm.at[pl.ds(base, nrow_t)], idx_v)

        def body(g, _):
            pltpu.sync_copy(ones_v, acc_sh.at[idx_v.at[g]], add=True)
            return 0

        lax.fori_loop(0, nrow_t, body, 0)
        plsc.subcore_barrier()
        pltpu.sync_copy(acc_sh.at[pl.ds(s * NZ, NZ)],
                        out_hbm.at[pl.ds(c * NN + s * NZ, NZ)])

    return _deg_kernel


@functools.cache
def _make_round(wc, npass):
    rows_t = ROWS_E // NTILE
    nplane = 2 * npass

    @functools.partial(
        pl.kernel,
        out_type=jax.ShapeDtypeStruct((nplane * NN, wc), jnp.float32),
        mesh=_sc_mesh(),
        scratch_types=[
            pltpu.VMEM((rows_t, CH), jnp.int32),
            pltpu.VMEM((rows_t, CH), jnp.int32),
            pltpu.VMEM((CH, wc), jnp.float32),
            pltpu.VMEM((CH, wc), jnp.float32),
            pltpu.VMEM_SHARED((NN, wc), jnp.float32),
            pltpu.SemaphoreType.DMA,
            pltpu.SemaphoreType.DMA,
            pltpu.SemaphoreType.DMA,
            pltpu.SemaphoreType.DMA,
        ],
        compiler_params=_SC_PARAMS,
    )
    def _round(srcb_hbm, dst_hbm, table_hbm, out_hbm, si_v, di_v, rows0_v,
               rows1_v, acc_sh, sem0, sem1, sems0, sems1):
        c = lax.axis_index("c")
        s = lax.axis_index("s")
        z16 = jnp.zeros((16,), jnp.float32)

        pltpu.sync_copy(dst_hbm.at[pl.ds(s * rows_t, rows_t)], di_v)
        for p in range(npass):
            plane = c * npass + p
            pltpu.sync_copy(srcb_hbm.at[plane, pl.ds(s * rows_t, rows_t)],
                            si_v)

            def zb_body(i, _):
                for j in range(wc // 16):
                    rows0_v[i, pl.ds(j * 16, 16)] = z16
                return 0

            lax.fori_loop(0, CH, zb_body, 0)
            for q in range(NZ // CH):
                pltpu.sync_copy(rows0_v, acc_sh.at[pl.ds(s * NZ + q * CH, CH)])
            plsc.subcore_barrier()

            pltpu.async_copy(table_hbm.at[si_v.at[0]], rows0_v, sem0)
            pltpu.async_copy(table_hbm.at[si_v.at[1]], rows1_v, sem1)

            def body(q, _):
                g0 = 2 * q
                g1 = g0 + 1
                g2 = jnp.minimum(g0 + 2, rows_t - 1)
                g3 = jnp.minimum(g0 + 3, rows_t - 1)
                pltpu.make_async_copy(table_hbm.at[si_v.at[g0]], rows0_v,
                                      sem0).wait()
                pltpu.async_copy(rows0_v, acc_sh.at[di_v.at[g0]], sems0,
                                 add=True)
                pltpu.make_async_copy(table_hbm.at[si_v.at[g1]], rows1_v,
                                      sem1).wait()
                pltpu.async_copy(rows1_v, acc_sh.at[di_v.at[g1]], sems1,
                                 add=True)
                pltpu.make_async_copy(rows0_v, acc_sh.at[di_v.at[g0]],
                                      sems0).wait()
                pltpu.async_copy(table_hbm.at[si_v.at[g2]], rows0_v, sem0)
                pltpu.make_async_copy(rows1_v, acc_sh.at[di_v.at[g1]],
                                      sems1).wait()
                pltpu.async_copy(table_hbm.at[si_v.at[g3]], rows1_v, sem1)
                return 0

            lax.fori_loop(0, rows_t // 2, body, 0)
            pltpu.make_async_copy(table_hbm.at[si_v.at[0]], rows0_v,
                                  sem0).wait()
            pltpu.make_async_copy(table_hbm.at[si_v.at[0]], rows1_v,
                                  sem1).wait()
            plsc.subcore_barrier()
            pltpu.sync_copy(acc_sh.at[pl.ds(s * NZ, NZ)],
                            out_hbm.at[pl.ds(plane * NN + s * NZ, NZ)])

    return _round


def _enc_call(x, mp):
    din = x.shape[1]

    def body(x_ref, w_ref, b_ref, a_ref, o_ref):
        h = jnp.dot(x_ref[...], w_ref[...],
                    preferred_element_type=jnp.float32) + b_ref[...]
        mu = jnp.mean(h, axis=0, keepdims=True)
        var = jnp.mean((h - mu) ** 2, axis=0, keepdims=True)
        hn = (h - mu) / jnp.sqrt(var + 1e-5)
        o_ref[...] = jnp.where(hn > 0, hn, a_ref[0, 0] * hn)

    return pl.pallas_call(
        body,
        out_shape=jax.ShapeDtypeStruct((NI, D), jnp.float32),
    )(x, mp["W"], mp["b"].reshape(1, D), jnp.reshape(mp["a"], (1, 1)))


def _prep2_call(ue_pad, et_pad, ev_pad, deg2, p):
    def body(ue_ref, et_ref, ev_ref, deg_ref,
             ew1, eb1, ew2, eb2, tw1, tb1, tw2, tb2, vw1, vb1, vw2, vb2,
             h0_ref, tab_ref, gd_ref, nrm_ref):
        deg = deg_ref[0, :, 0:1] + deg_ref[1, :, 0:1]
        nrm = lax.rsqrt(jnp.maximum(deg, 1.0))
        nrm_ref[...] = nrm
        xe = ue_ref[...]
        xt = et_ref[...]
        xv = ev_ref[...]
        h0 = jnp.concatenate([xe, xt, xv], axis=1)
        h0_ref[...] = h0
        hs = h0 * nrm
        for q in range(4):
            tab_ref[q] = hs[:, 96 * q:96 * (q + 1)]

        def gate(xg, w1, b1, w2, b2):
            l1 = jnp.maximum(
                jnp.dot(xg, w1[...], preferred_element_type=jnp.float32)
                + b1[...], 0.0)
            lg = jnp.dot(l1, w2[...],
                         preferred_element_type=jnp.float32) + b2[...]
            m = jnp.max(lg, axis=1, keepdims=True)
            e = jnp.exp(lg - m)
            return e / jnp.sum(e, axis=1, keepdims=True)

        we = gate(xe, ew1, eb1, ew2, eb2)
        wt = gate(xt, tw1, tb1, tw2, tb2)
        wv = gate(xv, vw1, vb1, vw2, vb2)
        gd_ref[...] = jnp.concatenate([we, wt, wv, deg], axis=1)

    eg, tg, vg = p["emb_gate"], p["t_gate"], p["v_gate"]
    full = lambda *shape: [
        pl.BlockSpec(shape, lambda i: tuple(0 for _ in shape))][0]
    return pl.pallas_call(
        body,
        grid=(NN // _UB,),
        in_specs=[
            pl.BlockSpec((_UB, D), lambda i: (i, 0)),
            pl.BlockSpec((_UB, D), lambda i: (i, 0)),
            pl.BlockSpec((_UB, D), lambda i: (i, 0)),
            pl.BlockSpec((2, _UB, 16), lambda i: (0, i, 0)),
            full(D, D), full(1, D), full(D, 3), full(1, 3),
            full(D, D), full(1, D), full(D, 2), full(1, 2),
            full(D, D), full(1, D), full(D, 2), full(1, 2),
        ],
        out_specs=[
            pl.BlockSpec((_UB, 3 * D), lambda i: (i, 0)),
            pl.BlockSpec((4, _UB, 96), lambda i: (0, i, 0)),
            pl.BlockSpec((_UB, 8), lambda i: (i, 0)),
            pl.BlockSpec((_UB, 1), lambda i: (i, 0)),
        ],
        out_shape=[
            jax.ShapeDtypeStruct((NN, 3 * D), jnp.float32),
            jax.ShapeDtypeStruct((4, NN, 96), jnp.float32),
            jax.ShapeDtypeStruct((NN, 8), jnp.float32),
            jax.ShapeDtypeStruct((NN, 1), jnp.float32),
        ],
    )(ue_pad, et_pad, ev_pad, deg2,
      eg["W1"], eg["b1"].reshape(1, D), eg["W2"], eg["b2"].reshape(1, 3),
      tg["W1"], tg["b1"].reshape(1, D), tg["W2"], tg["b2"].reshape(1, 2),
      vg["W1"], vg["b1"].reshape(1, D), vg["W2"], vg["b2"].reshape(1, 2))


_UB = 2000


def _u1_call(agg, h0, nrm):
    def body(agg_ref, h0_ref, nrm_ref, h_ref, tab_ref):
        nv = nrm_ref[...]
        a = jnp.concatenate([agg_ref[p] for p in range(4)], axis=1)
        h = ALPHA * nv * a + BETA * h0_ref[...]
        h_ref[...] = h
        hs = h * nv
        for p in range(4):
            tab_ref[p] = hs[:, 96 * p:96 * (p + 1)]

    return pl.pallas_call(
        body,
        grid=(NN // _UB,),
        in_specs=[
            pl.BlockSpec((4, _UB, 96), lambda i: (0, i, 0)),
            pl.BlockSpec((_UB, 3 * D), lambda i: (i, 0)),
            pl.BlockSpec((_UB, 1), lambda i: (i, 0)),
        ],
        out_specs=[
            pl.BlockSpec((_UB, 3 * D), lambda i: (i, 0)),
            pl.BlockSpec((4, _UB, 96), lambda i: (0, i, 0)),
        ],
        out_shape=[
            jax.ShapeDtypeStruct((NN, 3 * D), jnp.float32),
            jax.ShapeDtypeStruct((4, NN, 96), jnp.float32),
        ],
    )(agg, h0, nrm)


def _u2_call(agg, h0, nrm):
    def body(agg_ref, h0_ref, nrm_ref, h_ref, tab_ref):
        nv = nrm_ref[...]
        a = jnp.concatenate([agg_ref[p] for p in range(4)], axis=1)
        h = ALPHA * nv * a + BETA * h0_ref[...]
        h_ref[...] = h
        hs = h[:, 0:D] * nv
        tab_ref[0] = hs[:, 0:64]
        tab_ref[1] = hs[:, 64:128]

    return pl.pallas_call(
        body,
        grid=(NN // _UB,),
        in_specs=[
            pl.BlockSpec((4, _UB, 96), lambda i: (0, i, 0)),
            pl.BlockSpec((_UB, 3 * D), lambda i: (i, 0)),
            pl.BlockSpec((_UB, 1), lambda i: (i, 0)),
        ],
        out_specs=[
            pl.BlockSpec((_UB, 3 * D), lambda i: (i, 0)),
            pl.BlockSpec((2, _UB, 64), lambda i: (0, i, 0)),
        ],
        out_shape=[
            jax.ShapeDtypeStruct((NN, 3 * D), jnp.float32),
            jax.ShapeDtypeStruct((2, NN, 64), jnp.float32),
        ],
    )(agg, h0, nrm)


def _u3_call(agg, h0, nrm):
    def body(agg_ref, h0_ref, nrm_ref, h_ref):
        nv = nrm_ref[...]
        a = jnp.concatenate([agg_ref[0], agg_ref[1]], axis=1)
        h_ref[...] = ALPHA * nv * a + BETA * h0_ref[:, 0:D]

    return pl.pallas_call(
        body,
        grid=(NN // _UB,),
        in_specs=[
            pl.BlockSpec((2, _UB, 64), lambda i: (0, i, 0)),
            pl.BlockSpec((_UB, 3 * D), lambda i: (i, 0)),
            pl.BlockSpec((_UB, 1), lambda i: (i, 0)),
        ],
        out_specs=pl.BlockSpec((_UB, D), lambda i: (i, 0)),
        out_shape=jax.ShapeDtypeStruct((NN, D), jnp.float32),
    )(agg, h0, nrm)


def _fuse_a_call(h1, h2, h3, gd, p):
    import math as _math

    def body(h1r, h2r, h3r, gdr, w1e, w1t, w1v, w1c, b1, w2, b2,
             iwr, ibr, cscr, comb_ref, scol_ref):
        w = gdr[...]
        eh = (w[:, 0:1] * h1r[:, 0:D] + w[:, 1:2] * h2r[:, 0:D]
              + w[:, 2:3] * h3r[...])
        th = w[:, 3:4] * h1r[:, D:2 * D] + w[:, 4:5] * h2r[:, D:2 * D]
        vh = w[:, 5:6] * h1r[:, 2 * D:3 * D] + w[:, 6:7] * h2r[:, 2 * D:3 * D]

        def rown(x):
            return jnp.maximum(
                jnp.sqrt(jnp.sum(x * x, axis=1, keepdims=True)), 1e-8)

        ne_, nt_, nv_ = rown(eh), rown(th), rown(vh)
        cet = jnp.sum(eh * th, axis=1, keepdims=True) / (ne_ * nt_)
        cev = jnp.sum(eh * vh, axis=1, keepdims=True) / (ne_ * nv_)
        ctv = jnp.sum(th * vh, axis=1, keepdims=True) / (nt_ * nv_)
        conflict = jnp.concatenate([1.0 - cet, 1.0 - cev, 1.0 - ctv], axis=1)
        hin = jnp.maximum(
            jnp.dot(eh, w1e[...], preferred_element_type=jnp.float32)
            + jnp.dot(th, w1t[...], preferred_element_type=jnp.float32)
            + jnp.dot(vh, w1v[...], preferred_element_type=jnp.float32)
            + jnp.dot(conflict, w1c[...], preferred_element_type=jnp.float32)
            + b1[...], 0.0)
        logits = jnp.dot(hin, w2[...],
                         preferred_element_type=jnp.float32) + b2[...]
        msk011 = jnp.where(
            lax.broadcasted_iota(jnp.int32, (1, 3), 1) > 0, 1.0, 0.0)
        logits = logits - cscr[0, 0] * (1.0 - ctv) * msk011
        m = jnp.max(logits, axis=1, keepdims=True)
        e = jnp.exp(logits - m)
        mw = e / jnp.sum(e, axis=1, keepdims=True)
        comb = mw[:, 0:1] * eh + mw[:, 1:2] * th + mw[:, 2:3] * vh
        comb_ref[...] = comb

        def ent(pm, k):
            pc = jnp.maximum(pm, 1e-9)
            return (-jnp.sum(pc * jnp.log(pc), axis=1, keepdims=True)
                    / (_math.log(float(k)) + 1e-9))

        f_unc = ent(mw, 3)
        hop = (ent(w[:, 0:3], 3) + ent(w[:, 3:5], 2) + ent(w[:, 5:7], 2)) / 3.0
        unc = 0.5 * f_unc + 0.5 * hop
        imp = jnp.sum(comb * iwr[...], axis=1, keepdims=True) + ibr[0, 0]
        scol_ref[...] = jnp.concatenate([unc, imp, w[:, 7:8], unc], axis=1)

    f = p["fusion"]
    w1 = f["W1"]
    full = lambda *shape: [
        pl.BlockSpec(shape, lambda i: tuple(0 for _ in shape))][0]
    return pl.pallas_call(
        body,
        grid=(NN // _UB,),
        in_specs=[
            pl.BlockSpec((_UB, 3 * D), lambda i: (i, 0)),
            pl.BlockSpec((_UB, 3 * D), lambda i: (i, 0)),
            pl.BlockSpec((_UB, D), lambda i: (i, 0)),
            pl.BlockSpec((_UB, 8), lambda i: (i, 0)),
            full(D, D), full(D, D), full(D, D), full(3, D),
            full(1, D), full(D, 3), full(1, 3),
            full(1, D), full(1, 1), full(1, 1),
        ],
        out_specs=[
            pl.BlockSpec((_UB, D), lambda i: (i, 0)),
            pl.BlockSpec((_UB, 4), lambda i: (i, 0)),
        ],
        out_shape=[
            jax.ShapeDtypeStruct((NN, D), jnp.float32),
            jax.ShapeDtypeStruct((NN, 4), jnp.float32),
        ],
    )(h1, h2, h3, gd,
      w1[0:D], w1[D:2 * D], w1[2 * D:3 * D], w1[3 * D:3 * D + 3],
      f["b1"].reshape(1, D), f["W2"], f["b2"].reshape(1, 3),
      p["imp"]["W"].reshape(1, D), jnp.reshape(p["imp"]["b"], (1, 1)),
      jnp.reshape(p["conflict_scale"], (1, 1)))


def _fuse_b_call(comb_items, scol_items, p):
    def body(ihr, scr, coefr, gtok_ref):
        ih = ihr[...]
        scol = scr[...]
        center = jnp.mean(ih, axis=0, keepdims=True)
        cn = jnp.maximum(jnp.sqrt(jnp.sum(center * center)), 1e-8)
        inorm = jnp.maximum(
            jnp.sqrt(jnp.sum(ih * ih, axis=1, keepdims=True)), 1e-8)
        represent = jnp.sum(ih * center, axis=1, keepdims=True) / (inorm * cn)
        unc = scol[:, 0:1]
        imp = scol[:, 1:2]
        dcol = scol[:, 2:3]
        d = (dcol - jnp.min(dcol)) / (jnp.max(dcol) - jnp.min(dcol) + 1e-9)
        cf = coefr[...]
        sp = jnp.maximum(cf, 0.0) + jnp.log(1.0 + jnp.exp(-jnp.abs(cf)))
        score = (sp[0, 0] * represent + sp[0, 1] * unc + sp[0, 2] * d
                 + sp[0, 3] * imp)
        iota = lax.broadcasted_iota(jnp.int32, (NI, 1), 0)
        sc = score
        for j in range(4):
            mx = jnp.max(sc)
            ii = jnp.where(sc == mx, iota, NI)
            first = iota == jnp.min(ii)
            row = jnp.sum(jnp.where(first, 1.0, 0.0) * ih, axis=0,
                          keepdims=True)
            gtok_ref[pl.ds(j, 1), :] = row
            sc = jnp.where(first, -3e38, sc)

    return pl.pallas_call(
        body,
        out_shape=jax.ShapeDtypeStruct((4, D), jnp.float32),
    )(comb_items, scol_items, p["score_coef"].reshape(1, 4))


def kernel(user_embeddings, item_v_feat, item_t_feat, edge_index, params):
    src = edge_index[0]
    dst = edge_index[1]
    dst2 = dst.reshape(ROWS_E, CH)
    srcb = jnp.stack(
        [src, src + NN, src + 2 * NN, src + 3 * NN]).reshape(4, ROWS_E, CH)

    degp = _get_deg_kernel()(dst2)
    deg2 = degp.reshape(2, NN, 16)

    enc_t = _enc_call(item_t_feat, params["t_mlp"])
    enc_v = _enc_call(item_v_feat, params["v_mlp"])
    zu = jnp.zeros((NU, D), jnp.float32)
    zi = jnp.zeros((NI, D), jnp.float32)
    ue_pad = jnp.concatenate([user_embeddings, zi], axis=0)
    et_pad = jnp.concatenate([zu, enc_t], axis=0)
    ev_pad = jnp.concatenate([zu, enc_v], axis=0)
    h0, tab1, gd, nrm = _prep2_call(ue_pad, et_pad, ev_pad, deg2, params)

    agg1 = _make_round(96, 2)(srcb, dst2, tab1.reshape(4 * NN, 96))
    h1, tab2 = _u1_call(agg1.reshape(4, NN, 96), h0, nrm)
    agg2 = _make_round(96, 2)(srcb, dst2, tab2.reshape(4 * NN, 96))
    h2, tab3 = _u2_call(agg2.reshape(4, NN, 96), h0, nrm)
    agg3 = _make_round(64, 1)(srcb, dst2, tab3.reshape(2 * NN, 64))
    h3 = _u3_call(agg3.reshape(2, NN, 64), h0, nrm)

    combined, scol = _fuse_a_call(h1, h2, h3, gd, params)
    gtok = _fuse_b_call(combined[NU:], scol[NU:], params)

    item_h = combined[NU:]
    item_mem = jnp.concatenate(
        [item_h[:, None, :],
         jnp.broadcast_to(gtok[None], (NI, 4, D))], axis=1)
    z_mem = jnp.concatenate(
        [jnp.zeros((NU, 5, D), jnp.float32), item_mem], axis=0)
    return combined, z_mem


# speedup vs baseline: 1.3693x; 1.3693x over previous
import functools

import jax
import jax.numpy as jnp
from jax import lax
from jax.experimental import pallas as pl
from jax.experimental.pallas import tpu as pltpu
from jax.experimental.pallas import tpu_sc as plsc

NU = 6000
NI = 4000
NN = 10000
NE = 320000
D = 128
ALPHA = 0.8
BETA = 1.0

CH = 125
ROWS_E = NE // CH
NTILE = 16
NZ = NN // NTILE


def _sc_mesh():
    return plsc.VectorSubcoreMesh(core_axis_name="c", subcore_axis_name="s")


_SC_PARAMS = pltpu.CompilerParams(use_tc_tiling_on_sc=False)


@functools.cache
def _get_deg_kernel():
    @functools.partial(
        pl.kernel,
        out_type=jax.ShapeDtypeStruct((2 * NN, 16), jnp.float32),
        mesh=_sc_mesh(),
        scratch_types=[
            pltpu.VMEM((ROWS_E // 32, CH), jnp.int32),
            pltpu.VMEM((CH, 16), jnp.float32),
            pltpu.VMEM((NZ, 16), jnp.float32),
            pltpu.VMEM_SHARED((NN, 16), jnp.float32),
        ],
        compiler_params=_SC_PARAMS,
    )
    def _deg_kernel(dst_hbm, out_hbm, idx_v, ones_v, zb_v, acc_sh):
        c = lax.axis_index("c")
        s = lax.axis_index("s")
        nrow_t = ROWS_E // 32
        z16 = jnp.zeros((16,), jnp.float32)
        o16 = jnp.full((16,), 1.0, jnp.float32)

        def zb_body(i, _):
            zb_v[i, :] = z16
            return 0

        lax.fori_loop(0, NZ, zb_body, 0)

        def on_body(i, _):
            ones_v[i, :] = o16
            return 0

        lax.fori_loop(0, CH, on_body, 0)
        pltpu.sync_copy(zb_v, acc_sh.at[pl.ds(s * NZ, NZ)])
        plsc.subcore_barrier()

        base = (c * NTILE + s) * nrow_t
        pltpu.sync_copy(dst_hbm.at[pl.ds(base, nrow_t)], idx_v)

        def body(g, _):
            pltpu.sync_copy(ones_v, acc_sh.at[idx_v.at[g]], add=True)
            return 0

        lax.fori_loop(0, nrow_t, body, 0)
        plsc.subcore_barrier()
        pltpu.sync_copy(acc_sh.at[pl.ds(s * NZ, NZ)],
                        out_hbm.at[pl.ds(c * NN + s * NZ, NZ)])

    return _deg_kernel


@functools.cache
def _make_round(wc, npass):
    rows_t = ROWS_E // NTILE
    nplane = 2 * npass

    half_t = rows_t // 2

    @functools.partial(
        pl.kernel,
        out_type=jax.ShapeDtypeStruct((nplane * NN, wc), jnp.float32),
        mesh=_sc_mesh(),
        scratch_types=[
            pltpu.VMEM((half_t, CH), jnp.int32),
            pltpu.VMEM((half_t, CH), jnp.int32),
            pltpu.VMEM((CH, wc), jnp.float32),
            pltpu.VMEM((CH, wc), jnp.float32),
            pltpu.VMEM((CH, wc), jnp.float32),
            pltpu.VMEM((CH, wc), jnp.float32),
            pltpu.VMEM_SHARED((NN, wc), jnp.float32),
            pltpu.SemaphoreType.DMA,
            pltpu.SemaphoreType.DMA,
            pltpu.SemaphoreType.DMA,
            pltpu.SemaphoreType.DMA,
        ],
        compiler_params=_SC_PARAMS,
    )
    def _round(srcb_hbm, dst_hbm, table_hbm, out_hbm, si_v, di_v, rows0_v,
               rows1_v, rows2_v, rows3_v, acc_sh, sem0, sem1, sem2, sem3):
        c = lax.axis_index("c")
        s = lax.axis_index("s")
        z16 = jnp.zeros((16,), jnp.float32)
        bufs = (rows0_v, rows1_v, rows2_v, rows3_v)
        sems = (sem0, sem1, sem2, sem3)

        def zb_body(i, _):
            for j in range(wc // 16):
                rows0_v[i, pl.ds(j * 16, 16)] = z16
            return 0

        for p in range(npass):
            plane = c * npass + p

            lax.fori_loop(0, CH, zb_body, 0)
            for q in range(NZ // CH):
                pltpu.sync_copy(rows0_v, acc_sh.at[pl.ds(s * NZ + q * CH, CH)])
            plsc.subcore_barrier()

            for half in range(2):
                base = s * rows_t + half * half_t
                pltpu.sync_copy(srcb_hbm.at[plane, pl.ds(base, half_t)],
                                si_v)
                pltpu.sync_copy(dst_hbm.at[pl.ds(base, half_t)], di_v)

                for k in range(3):
                    pltpu.async_copy(table_hbm.at[si_v.at[k]], bufs[k],
                                     sems[k])

                def body(q, _):
                    for k in range(4):
                        g = 4 * q + k
                        gnext = jnp.minimum(g + 3, half_t - 1)
                        pltpu.make_async_copy(table_hbm.at[si_v.at[g]],
                                              bufs[k], sems[k]).wait()
                        pltpu.sync_copy(bufs[k], acc_sh.at[di_v.at[g]],
                                        add=True)
                        pltpu.async_copy(table_hbm.at[si_v.at[gnext]],
                                         bufs[(k + 3) % 4],
                                         sems[(k + 3) % 4])
                    return 0

                lax.fori_loop(0, half_t // 4, body, 0)
                for k in range(3):
                    pltpu.make_async_copy(table_hbm.at[si_v.at[0]],
                                          bufs[k], sems[k]).wait()

            plsc.subcore_barrier()
            pltpu.sync_copy(acc_sh.at[pl.ds(s * NZ, NZ)],
                            out_hbm.at[pl.ds(plane * NN + s * NZ, NZ)])

    return _round


def _enc_call(x, mp):
    din = x.shape[1]

    def body(x_ref, w_ref, b_ref, a_ref, o_ref):
        h = jnp.dot(x_ref[...], w_ref[...],
                    preferred_element_type=jnp.float32) + b_ref[...]
        mu = jnp.mean(h, axis=0, keepdims=True)
        var = jnp.mean((h - mu) ** 2, axis=0, keepdims=True)
        hn = (h - mu) / jnp.sqrt(var + 1e-5)
        o_ref[...] = jnp.where(hn > 0, hn, a_ref[0, 0] * hn)

    return pl.pallas_call(
        body,
        out_shape=jax.ShapeDtypeStruct((NI, D), jnp.float32),
    )(x, mp["W"], mp["b"].reshape(1, D), jnp.reshape(mp["a"], (1, 1)))


def _prep2_call(ue_pad, et_pad, ev_pad, deg2, p):
    def body(ue_ref, et_ref, ev_ref, deg_ref,
             ew1, eb1, ew2, eb2, tw1, tb1, tw2, tb2, vw1, vb1, vw2, vb2,
             h0_ref, tab_ref, gd_ref, nrm_ref):
        deg = deg_ref[0, :, 0:1] + deg_ref[1, :, 0:1]
        nrm = lax.rsqrt(jnp.maximum(deg, 1.0))
        nrm_ref[...] = nrm
        xe = ue_ref[...]
        xt = et_ref[...]
        xv = ev_ref[...]
        h0 = jnp.concatenate([xe, xt, xv], axis=1)
        h0_ref[...] = h0
        hs = h0 * nrm
        for q in range(4):
            tab_ref[q] = hs[:, 96 * q:96 * (q + 1)]

        def gate(xg, w1, b1, w2, b2):
            l1 = jnp.maximum(
                jnp.dot(xg, w1[...], preferred_element_type=jnp.float32)
                + b1[...], 0.0)
            lg = jnp.dot(l1, w2[...],
                         preferred_element_type=jnp.float32) + b2[...]
            m = jnp.max(lg, axis=1, keepdims=True)
            e = jnp.exp(lg - m)
            return e / jnp.sum(e, axis=1, keepdims=True)

        we = gate(xe, ew1, eb1, ew2, eb2)
        wt = gate(xt, tw1, tb1, tw2, tb2)
        wv = gate(xv, vw1, vb1, vw2, vb2)
        gd_ref[...] = jnp.concatenate([we, wt, wv, deg], axis=1)

    eg, tg, vg = p["emb_gate"], p["t_gate"], p["v_gate"]
    full = lambda *shape: [
        pl.BlockSpec(shape, lambda i: tuple(0 for _ in shape))][0]
    return pl.pallas_call(
        body,
        grid=(NN // _UB,),
        in_specs=[
            pl.BlockSpec((_UB, D), lambda i: (i, 0)),
            pl.BlockSpec((_UB, D), lambda i: (i, 0)),
            pl.BlockSpec((_UB, D), lambda i: (i, 0)),
            pl.BlockSpec((2, _UB, 16), lambda i: (0, i, 0)),
            full(D, D), full(1, D), full(D, 3), full(1, 3),
            full(D, D), full(1, D), full(D, 2), full(1, 2),
            full(D, D), full(1, D), full(D, 2), full(1, 2),
        ],
        out_specs=[
            pl.BlockSpec((_UB, 3 * D), lambda i: (i, 0)),
            pl.BlockSpec((4, _UB, 96), lambda i: (0, i, 0)),
            pl.BlockSpec((_UB, 8), lambda i: (i, 0)),
            pl.BlockSpec((_UB, 1), lambda i: (i, 0)),
        ],
        out_shape=[
            jax.ShapeDtypeStruct((NN, 3 * D), jnp.float32),
            jax.ShapeDtypeStruct((4, NN, 96), jnp.float32),
            jax.ShapeDtypeStruct((NN, 8), jnp.float32),
            jax.ShapeDtypeStruct((NN, 1), jnp.float32),
        ],
    )(ue_pad, et_pad, ev_pad, deg2,
      eg["W1"], eg["b1"].reshape(1, D), eg["W2"], eg["b2"].reshape(1, 3),
      tg["W1"], tg["b1"].reshape(1, D), tg["W2"], tg["b2"].reshape(1, 2),
      vg["W1"], vg["b1"].reshape(1, D), vg["W2"], vg["b2"].reshape(1, 2))


_UB = 2000


def _u1_call(agg, h0, nrm):
    def body(agg_ref, h0_ref, nrm_ref, h_ref, tab_ref):
        nv = nrm_ref[...]
        a = jnp.concatenate([agg_ref[p] for p in range(4)], axis=1)
        h = ALPHA * nv * a + BETA * h0_ref[...]
        h_ref[...] = h
        hs = h * nv
        for p in range(4):
            tab_ref[p] = hs[:, 96 * p:96 * (p + 1)]

    return pl.pallas_call(
        body,
        grid=(NN // _UB,),
        in_specs=[
            pl.BlockSpec((4, _UB, 96), lambda i: (0, i, 0)),
            pl.BlockSpec((_UB, 3 * D), lambda i: (i, 0)),
            pl.BlockSpec((_UB, 1), lambda i: (i, 0)),
        ],
        out_specs=[
            pl.BlockSpec((_UB, 3 * D), lambda i: (i, 0)),
            pl.BlockSpec((4, _UB, 96), lambda i: (0, i, 0)),
        ],
        out_shape=[
            jax.ShapeDtypeStruct((NN, 3 * D), jnp.float32),
            jax.ShapeDtypeStruct((4, NN, 96), jnp.float32),
        ],
    )(agg, h0, nrm)


def _u2_call(agg, h0, nrm):
    def body(agg_ref, h0_ref, nrm_ref, h_ref, tab_ref):
        nv = nrm_ref[...]
        a = jnp.concatenate([agg_ref[p] for p in range(4)], axis=1)
        h = ALPHA * nv * a + BETA * h0_ref[...]
        h_ref[...] = h
        hs = h[:, 0:D] * nv
        tab_ref[0] = hs[:, 0:64]
        tab_ref[1] = hs[:, 64:128]

    return pl.pallas_call(
        body,
        grid=(NN // _UB,),
        in_specs=[
            pl.BlockSpec((4, _UB, 96), lambda i: (0, i, 0)),
            pl.BlockSpec((_UB, 3 * D), lambda i: (i, 0)),
            pl.BlockSpec((_UB, 1), lambda i: (i, 0)),
        ],
        out_specs=[
            pl.BlockSpec((_UB, 3 * D), lambda i: (i, 0)),
            pl.BlockSpec((2, _UB, 64), lambda i: (0, i, 0)),
        ],
        out_shape=[
            jax.ShapeDtypeStruct((NN, 3 * D), jnp.float32),
            jax.ShapeDtypeStruct((2, NN, 64), jnp.float32),
        ],
    )(agg, h0, nrm)


def _u3_call(agg, h0, nrm):
    def body(agg_ref, h0_ref, nrm_ref, h_ref):
        nv = nrm_ref[...]
        a = jnp.concatenate([agg_ref[0], agg_ref[1]], axis=1)
        h_ref[...] = ALPHA * nv * a + BETA * h0_ref[:, 0:D]

    return pl.pallas_call(
        body,
        grid=(NN // _UB,),
        in_specs=[
            pl.BlockSpec((2, _UB, 64), lambda i: (0, i, 0)),
            pl.BlockSpec((_UB, 3 * D), lambda i: (i, 0)),
            pl.BlockSpec((_UB, 1), lambda i: (i, 0)),
        ],
        out_specs=pl.BlockSpec((_UB, D), lambda i: (i, 0)),
        out_shape=jax.ShapeDtypeStruct((NN, D), jnp.float32),
    )(agg, h0, nrm)


def _fuse_a_call(h1, h2, h3, gd, p):
    import math as _math

    def body(h1r, h2r, h3r, gdr, w1e, w1t, w1v, w1c, b1, w2, b2,
             iwr, ibr, cscr, comb_ref, scol_ref):
        w = gdr[...]
        eh = (w[:, 0:1] * h1r[:, 0:D] + w[:, 1:2] * h2r[:, 0:D]
              + w[:, 2:3] * h3r[...])
        th = w[:, 3:4] * h1r[:, D:2 * D] + w[:, 4:5] * h2r[:, D:2 * D]
        vh = w[:, 5:6] * h1r[:, 2 * D:3 * D] + w[:, 6:7] * h2r[:, 2 * D:3 * D]

        def rown(x):
            return jnp.maximum(
                jnp.sqrt(jnp.sum(x * x, axis=1, keepdims=True)), 1e-8)

        ne_, nt_, nv_ = rown(eh), rown(th), rown(vh)
        cet = jnp.sum(eh * th, axis=1, keepdims=True) / (ne_ * nt_)
        cev = jnp.sum(eh * vh, axis=1, keepdims=True) / (ne_ * nv_)
        ctv = jnp.sum(th * vh, axis=1, keepdims=True) / (nt_ * nv_)
        conflict = jnp.concatenate([1.0 - cet, 1.0 - cev, 1.0 - ctv], axis=1)
        hin = jnp.maximum(
            jnp.dot(eh, w1e[...], preferred_element_type=jnp.float32)
            + jnp.dot(th, w1t[...], preferred_element_type=jnp.float32)
            + jnp.dot(vh, w1v[...], preferred_element_type=jnp.float32)
            + jnp.dot(conflict, w1c[...], preferred_element_type=jnp.float32)
            + b1[...], 0.0)
        logits = jnp.dot(hin, w2[...],
                         preferred_element_type=jnp.float32) + b2[...]
        msk011 = jnp.where(
            lax.broadcasted_iota(jnp.int32, (1, 3), 1) > 0, 1.0, 0.0)
        logits = logits - cscr[0, 0] * (1.0 - ctv) * msk011
        m = jnp.max(logits, axis=1, keepdims=True)
        e = jnp.exp(logits - m)
        mw = e / jnp.sum(e, axis=1, keepdims=True)
        comb = mw[:, 0:1] * eh + mw[:, 1:2] * th + mw[:, 2:3] * vh
        comb_ref[...] = comb

        def ent(pm, k):
            pc = jnp.maximum(pm, 1e-9)
            return (-jnp.sum(pc * jnp.log(pc), axis=1, keepdims=True)
                    / (_math.log(float(k)) + 1e-9))

        f_unc = ent(mw, 3)
        hop = (ent(w[:, 0:3], 3) + ent(w[:, 3:5], 2) + ent(w[:, 5:7], 2)) / 3.0
        unc = 0.5 * f_unc + 0.5 * hop
        imp = jnp.sum(comb * iwr[...], axis=1, keepdims=True) + ibr[0, 0]
        scol_ref[...] = jnp.concatenate([unc, imp, w[:, 7:8], unc], axis=1)

    f = p["fusion"]
    w1 = f["W1"]
    full = lambda *shape: [
        pl.BlockSpec(shape, lambda i: tuple(0 for _ in shape))][0]
    return pl.pallas_call(
        body,
        grid=(NN // _UB,),
        in_specs=[
            pl.BlockSpec((_UB, 3 * D), lambda i: (i, 0)),
            pl.BlockSpec((_UB, 3 * D), lambda i: (i, 0)),
            pl.BlockSpec((_UB, D), lambda i: (i, 0)),
            pl.BlockSpec((_UB, 8), lambda i: (i, 0)),
            full(D, D), full(D, D), full(D, D), full(3, D),
            full(1, D), full(D, 3), full(1, 3),
            full(1, D), full(1, 1), full(1, 1),
        ],
        out_specs=[
            pl.BlockSpec((_UB, D), lambda i: (i, 0)),
            pl.BlockSpec((_UB, 4), lambda i: (i, 0)),
        ],
        out_shape=[
            jax.ShapeDtypeStruct((NN, D), jnp.float32),
            jax.ShapeDtypeStruct((NN, 4), jnp.float32),
        ],
    )(h1, h2, h3, gd,
      w1[0:D], w1[D:2 * D], w1[2 * D:3 * D], w1[3 * D:3 * D + 3],
      f["b1"].reshape(1, D), f["W2"], f["b2"].reshape(1, 3),
      p["imp"]["W"].reshape(1, D), jnp.reshape(p["imp"]["b"], (1, 1)),
      jnp.reshape(p["conflict_scale"], (1, 1)))


def _fuse_b_call(comb_items, scol_items, p):
    def body(ihr, scr, coefr, gtok_ref):
        ih = ihr[...]
        scol = scr[...]
        center = jnp.mean(ih, axis=0, keepdims=True)
        cn = jnp.maximum(jnp.sqrt(jnp.sum(center * center)), 1e-8)
        inorm = jnp.maximum(
            jnp.sqrt(jnp.sum(ih * ih, axis=1, keepdims=True)), 1e-8)
        represent = jnp.sum(ih * center, axis=1, keepdims=True) / (inorm * cn)
        unc = scol[:, 0:1]
        imp = scol[:, 1:2]
        dcol = scol[:, 2:3]
        d = (dcol - jnp.min(dcol)) / (jnp.max(dcol) - jnp.min(dcol) + 1e-9)
        cf = coefr[...]
        sp = jnp.maximum(cf, 0.0) + jnp.log(1.0 + jnp.exp(-jnp.abs(cf)))
        score = (sp[0, 0] * represent + sp[0, 1] * unc + sp[0, 2] * d
                 + sp[0, 3] * imp)
        iota = lax.broadcasted_iota(jnp.int32, (NI, 1), 0)
        sc = score
        for j in range(4):
            mx = jnp.max(sc)
            ii = jnp.where(sc == mx, iota, NI)
            first = iota == jnp.min(ii)
            row = jnp.sum(jnp.where(first, 1.0, 0.0) * ih, axis=0,
                          keepdims=True)
            gtok_ref[pl.ds(j, 1), :] = row
            sc = jnp.where(first, -3e38, sc)

    return pl.pallas_call(
        body,
        out_shape=jax.ShapeDtypeStruct((4, D), jnp.float32),
    )(comb_items, scol_items, p["score_coef"].reshape(1, 4))


def kernel(user_embeddings, item_v_feat, item_t_feat, edge_index, params):
    src = edge_index[0]
    dst = edge_index[1]
    dst2 = dst.reshape(ROWS_E, CH)
    srcb = jnp.stack(
        [src, src + NN, src + 2 * NN, src + 3 * NN]).reshape(4, ROWS_E, CH)

    degp = _get_deg_kernel()(dst2)
    deg2 = degp.reshape(2, NN, 16)

    enc_t = _enc_call(item_t_feat, params["t_mlp"])
    enc_v = _enc_call(item_v_feat, params["v_mlp"])
    zu = jnp.zeros((NU, D), jnp.float32)
    zi = jnp.zeros((NI, D), jnp.float32)
    ue_pad = jnp.concatenate([user_embeddings, zi], axis=0)
    et_pad = jnp.concatenate([zu, enc_t], axis=0)
    ev_pad = jnp.concatenate([zu, enc_v], axis=0)
    h0, tab1, gd, nrm = _prep2_call(ue_pad, et_pad, ev_pad, deg2, params)

    agg1 = _make_round(96, 2)(srcb, dst2, tab1.reshape(4 * NN, 96))
    h1, tab2 = _u1_call(agg1.reshape(4, NN, 96), h0, nrm)
    agg2 = _make_round(96, 2)(srcb, dst2, tab2.reshape(4 * NN, 96))
    h2, tab3 = _u2_call(agg2.reshape(4, NN, 96), h0, nrm)
    agg3 = _make_round(64, 1)(srcb, dst2, tab3.reshape(2 * NN, 64))
    h3 = _u3_call(agg3.reshape(2, NN, 64), h0, nrm)

    combined, scol = _fuse_a_call(h1, h2, h3, gd, params)
    gtok = _fuse_b_call(combined[NU:], scol[NU:], params)

    item_h = combined[NU:]
    item_mem = jnp.concatenate(
        [item_h[:, None, :],
         jnp.broadcast_to(gtok[None], (NI, 4, D))], axis=1)
    z_mem = jnp.concatenate(
        [jnp.zeros((NU, 5, D), jnp.float32), item_mem], axis=0)
    return combined, z_mem
